# baseline (speedup 1.0000x reference)
import jax
import jax.numpy as jnp
from jax.experimental import pallas as pl
from jax.experimental.pallas import tpu as pltpu

_B, _S, _D = 1, 2048, 768
_E, _K, _F = 8, 2, 2048
_N = _B * _S
_TILE = 256
_NPAD = _N * _K + _E * _TILE
_NTILES = _NPAD // _TILE
_CH = 1024
_NCHUNK = _N // _CH
_RSROWS = 8


def _router_body(flat_ref, wg_ref,
                 rte_ref, comb_ref, padoff_ref, texp_ref, used_ref, rs_ref,
                 aux_ref):
    flat = flat_ref[...]
    logits = jnp.dot(flat, wg_ref[...], preferred_element_type=jnp.float32)

    eio = jax.lax.broadcasted_iota(jnp.int32, (_N, _E), 1)
    m1 = jnp.max(logits, axis=1, keepdims=True)
    i1 = jnp.argmax(logits, axis=1)[:, None]
    oh1 = (eio == i1).astype(jnp.float32)
    masked = jnp.where(oh1 > 0.0, jnp.float32(-1e30), logits)
    m2 = jnp.max(masked, axis=1, keepdims=True)
    i2 = jnp.argmax(masked, axis=1)[:, None]
    oh2 = (eio == i2).astype(jnp.float32)

    w1 = 1.0 / (1.0 + jnp.exp(m2 - m1))
    w2 = 1.0 - w1
    comb_ref[...] = w1 * oh1 + w2 * oh2

    p = jnp.exp(logits - m1)
    p = p / jnp.sum(p, axis=1, keepdims=True)
    sel = oh1 + oh2
    aux_ref[...] = (_E / (_N * _N * _K)) * jnp.sum(
        jnp.sum(p, axis=0, keepdims=True) * jnp.sum(sel, axis=0, keepdims=True),
        axis=1, keepdims=True)

    cnt = jnp.sum(sel, axis=0, keepdims=True)
    ch_r = jax.lax.broadcasted_iota(jnp.int32, (_RSROWS, _N), 0)
    ch_c = jax.lax.broadcasted_iota(jnp.int32, (_RSROWS, _N), 1)
    lb = (ch_c < ch_r * _CH).astype(jnp.float32)
    rs = jax.lax.dot_general(lb, sel, (((1,), (0,)), ((), ())),
                             preferred_element_type=jnp.float32)
    rs_ref[...] = rs.astype(jnp.int32)

    l_r = jax.lax.broadcasted_iota(jnp.int32, (_CH, _CH), 0)
    l_c = jax.lax.broadcasted_iota(jnp.int32, (_CH, _CH), 1)
    l256 = (l_c < l_r).astype(jnp.float32)
    excl_parts = []
    for c in range(_NCHUNK):
        sl = slice(c * _CH, (c + 1) * _CH)
        local = jax.lax.dot_general(l256, sel[sl, :], (((1,), (0,)), ((), ())),
                                    preferred_element_type=jnp.float32)
        excl_parts.append(local + rs[c:c + 1, :])
    excl = jnp.concatenate(excl_parts, axis=0)

    rte_ref[...] = jnp.where(sel > 0.0, excl, jnp.float32(-1.0))

    pc = jnp.ceil(cnt / _TILE) * _TILE
    e_r = jax.lax.broadcasted_iota(jnp.int32, (_E, _E), 0)
    e_c = jax.lax.broadcasted_iota(jnp.int32, (_E, _E), 1)
    ltri_e = (e_r < e_c).astype(jnp.float32)
    padoff = jax.lax.dot_general(pc, ltri_e, (((1,), (0,)), ((), ())),
                                 preferred_element_type=jnp.float32)
    pad_end = padoff + pc
    padoff_ref[...] = padoff.astype(jnp.int32)
    used_ref[...] = jnp.sum(pc, axis=1, keepdims=True).astype(jnp.int32)

    t_starts = jax.lax.broadcasted_iota(
        jnp.int32, (_NTILES, _E), 0).astype(jnp.float32) * _TILE
    texp = jnp.sum((t_starts >= pad_end).astype(jnp.float32),
                   axis=1, keepdims=True)
    texp_ref[...] = jnp.minimum(texp, _E - 1).astype(jnp.int32)


def _ffn_body(texp_ref, padoff_ref, used_ref, rs_ref,
              rte_ref, comb_ref, flat_ref, w1_ref, b1_ref, w2_ref, b2_ref,
              out_ref, gx_s, w1b_s, w2b_s):
    t = pl.program_id(0)

    @pl.when(t == 0)
    def _():
        out_ref[...] = jnp.zeros_like(out_ref)

    e = texp_ref[t]
    eprev = texp_ref[jnp.maximum(t - 1, 0)]

    @pl.when((t == 0) | (e != eprev))
    def _():
        w1b_s[...] = w1_ref[0].astype(jnp.bfloat16)
        w2b_s[...] = w2_ref[0].astype(jnp.bfloat16)

    @pl.when(t * _TILE < used_ref[0])
    def _():
        r0 = t * _TILE - padoff_ref[e]
        r0f = r0.astype(jnp.float32)
        lane = jax.lax.broadcasted_iota(
            jnp.int32, (1, _TILE), 1).astype(jnp.float32)

        gx_s[...] = jnp.zeros((_TILE, _D), jnp.float32)
        for c in range(_NCHUNK):
            lo = rs_ref[c * _E + e]
            hi = rs_ref[(c + 1) * _E + e]

            @pl.when((lo < r0 + _TILE) & (hi > r0))
            def _(c=c):
                sl = slice(c * _CH, (c + 1) * _CH)
                ei = jax.lax.broadcasted_iota(jnp.int32, (_CH, _E), 1)
                esel = (ei == e).astype(jnp.float32)
                rcol = jnp.sum(rte_ref[sl, :] * esel, axis=1, keepdims=True)
                mt = (rcol == r0f + lane).astype(jnp.float32)
                gx_s[...] += jax.lax.dot_general(
                    mt, flat_ref[sl, :], (((0,), (0,)), ((), ())),
                    preferred_element_type=jnp.float32)

        h = jax.nn.gelu(jnp.dot(gx_s[...].astype(jnp.bfloat16), w1b_s[...],
                                preferred_element_type=jnp.float32)
                        + b1_ref[0])
        y = jnp.dot(h.astype(jnp.bfloat16), w2b_s[...],
                    preferred_element_type=jnp.float32) + b2_ref[0]

        for c in range(_NCHUNK):
            lo = rs_ref[c * _E + e]
            hi = rs_ref[(c + 1) * _E + e]

            @pl.when((lo < r0 + _TILE) & (hi > r0))
            def _(c=c):
                sl = slice(c * _CH, (c + 1) * _CH)
                ei = jax.lax.broadcasted_iota(jnp.int32, (_CH, _E), 1)
                esel = (ei == e).astype(jnp.float32)
                rcol = jnp.sum(rte_ref[sl, :] * esel, axis=1, keepdims=True)
                wcol = jnp.sum(comb_ref[sl, :] * esel, axis=1, keepdims=True)
                mtw = (rcol == r0f + lane).astype(jnp.float32) * wcol
                out_ref[sl, :] += jnp.dot(mtw, y,
                                          preferred_element_type=jnp.float32)


@jax.jit
def kernel(x, Wg, W1, b1, W2, b2):
    flat = x.reshape(_N, _D)

    router = pl.pallas_call(
        _router_body,
        grid=(1,),
        in_specs=[
            pl.BlockSpec((_N, _D), lambda i: (0, 0)),
            pl.BlockSpec((_D, _E), lambda i: (0, 0)),
        ],
        out_specs=[
            pl.BlockSpec((_N, _E), lambda i: (0, 0)),
            pl.BlockSpec((_N, _E), lambda i: (0, 0)),
            pl.BlockSpec((1, _E), lambda i: (0, 0)),
            pl.BlockSpec((_NTILES, 1), lambda i: (0, 0)),
            pl.BlockSpec((1, 1), lambda i: (0, 0)),
            pl.BlockSpec((_RSROWS, _E), lambda i: (0, 0)),
            pl.BlockSpec((1, 1), lambda i: (0, 0)),
        ],
        out_shape=[
            jax.ShapeDtypeStruct((_N, _E), jnp.float32),
            jax.ShapeDtypeStruct((_N, _E), jnp.float32),
            jax.ShapeDtypeStruct((1, _E), jnp.int32),
            jax.ShapeDtypeStruct((_NTILES, 1), jnp.int32),
            jax.ShapeDtypeStruct((1, 1), jnp.int32),
            jax.ShapeDtypeStruct((_RSROWS, _E), jnp.int32),
            jax.ShapeDtypeStruct((1, 1), jnp.float32),
        ],
    )
    rte, comb, padoff, texp, used, rs, aux = router(flat, Wg)

    grid_spec = pltpu.PrefetchScalarGridSpec(
        num_scalar_prefetch=4,
        grid=(_NTILES,),
        in_specs=[
            pl.BlockSpec((_N, _E), lambda t, te, po, us, rs: (0, 0)),
            pl.BlockSpec((_N, _E), lambda t, te, po, us, rs: (0, 0)),
            pl.BlockSpec((_N, _D), lambda t, te, po, us, rs: (0, 0)),
            pl.BlockSpec((1, _D, _F), lambda t, te, po, us, rs: (te[t], 0, 0)),
            pl.BlockSpec((1, 1, _F), lambda t, te, po, us, rs: (te[t], 0, 0)),
            pl.BlockSpec((1, _F, _D), lambda t, te, po, us, rs: (te[t], 0, 0)),
            pl.BlockSpec((1, 1, _D), lambda t, te, po, us, rs: (te[t], 0, 0)),
        ],
        out_specs=pl.BlockSpec((_N, _D), lambda t, te, po, us, rs: (0, 0)),
        scratch_shapes=[pltpu.VMEM((_TILE, _D), jnp.float32),
                        pltpu.VMEM((_D, _F), jnp.bfloat16),
                        pltpu.VMEM((_F, _D), jnp.bfloat16)],
    )
    ffn = pl.pallas_call(
        _ffn_body,
        grid_spec=grid_spec,
        out_shape=jax.ShapeDtypeStruct((_N, _D), jnp.float32),
        compiler_params=pltpu.CompilerParams(
            dimension_semantics=("arbitrary",),
            vmem_limit_bytes=128 * 1024 * 1024),
    )
    out = ffn(texp.reshape(_NTILES), padoff.reshape(_E), used.reshape(1),
              rs.reshape(_RSROWS * _E),
              rte, comb, flat, W1, b1.reshape(_E, 1, _F), W2,
              b2.reshape(_E, 1, _D))

    return out.reshape(_B, _S, _D), aux[0, 0]


# speedup vs baseline: 1.0131x; 1.0131x over previous
import jax
import jax.numpy as jnp
from jax.experimental import pallas as pl
from jax.experimental.pallas import tpu as pltpu

_B, _S, _D = 1, 2048, 768
_E, _K, _F = 8, 2, 2048
_N = _B * _S
_TILE = 512
_NPAD = _N * _K + _E * _TILE
_NTILES = _NPAD // _TILE
_CH = 1024
_NCHUNK = _N // _CH
_RSROWS = 8


def _router_body(flat_ref, wg_ref,
                 rte_ref, comb_ref, padoff_ref, texp_ref, used_ref, rs_ref,
                 aux_ref):
    flat = flat_ref[...]
    logits = jnp.dot(flat, wg_ref[...], preferred_element_type=jnp.float32)

    eio = jax.lax.broadcasted_iota(jnp.int32, (_N, _E), 1)
    m1 = jnp.max(logits, axis=1, keepdims=True)
    i1 = jnp.argmax(logits, axis=1)[:, None]
    oh1 = (eio == i1).astype(jnp.float32)
    masked = jnp.where(oh1 > 0.0, jnp.float32(-1e30), logits)
    m2 = jnp.max(masked, axis=1, keepdims=True)
    i2 = jnp.argmax(masked, axis=1)[:, None]
    oh2 = (eio == i2).astype(jnp.float32)

    w1 = 1.0 / (1.0 + jnp.exp(m2 - m1))
    w2 = 1.0 - w1
    comb_ref[...] = w1 * oh1 + w2 * oh2

    p = jnp.exp(logits - m1)
    p = p / jnp.sum(p, axis=1, keepdims=True)
    sel = oh1 + oh2
    aux_ref[...] = (_E / (_N * _N * _K)) * jnp.sum(
        jnp.sum(p, axis=0, keepdims=True) * jnp.sum(sel, axis=0, keepdims=True),
        axis=1, keepdims=True)

    cnt = jnp.sum(sel, axis=0, keepdims=True)
    ch_r = jax.lax.broadcasted_iota(jnp.int32, (_RSROWS, _N), 0)
    ch_c = jax.lax.broadcasted_iota(jnp.int32, (_RSROWS, _N), 1)
    lb = (ch_c < ch_r * _CH).astype(jnp.float32)
    rs = jax.lax.dot_general(lb, sel, (((1,), (0,)), ((), ())),
                             preferred_element_type=jnp.float32)
    rs_ref[...] = rs.astype(jnp.int32)

    l_r = jax.lax.broadcasted_iota(jnp.int32, (_CH, _CH), 0)
    l_c = jax.lax.broadcasted_iota(jnp.int32, (_CH, _CH), 1)
    l256 = (l_c < l_r).astype(jnp.float32)
    excl_parts = []
    for c in range(_NCHUNK):
        sl = slice(c * _CH, (c + 1) * _CH)
        local = jax.lax.dot_general(l256, sel[sl, :], (((1,), (0,)), ((), ())),
                                    preferred_element_type=jnp.float32)
        excl_parts.append(local + rs[c:c + 1, :])
    excl = jnp.concatenate(excl_parts, axis=0)

    rte_ref[...] = jnp.where(sel > 0.0, excl, jnp.float32(-1.0))

    pc = jnp.ceil(cnt / _TILE) * _TILE
    e_r = jax.lax.broadcasted_iota(jnp.int32, (_E, _E), 0)
    e_c = jax.lax.broadcasted_iota(jnp.int32, (_E, _E), 1)
    ltri_e = (e_r < e_c).astype(jnp.float32)
    padoff = jax.lax.dot_general(pc, ltri_e, (((1,), (0,)), ((), ())),
                                 preferred_element_type=jnp.float32)
    pad_end = padoff + pc
    padoff_ref[...] = padoff.astype(jnp.int32)
    used_ref[...] = jnp.sum(pc, axis=1, keepdims=True).astype(jnp.int32)

    t_starts = jax.lax.broadcasted_iota(
        jnp.int32, (_NTILES, _E), 0).astype(jnp.float32) * _TILE
    texp = jnp.sum((t_starts >= pad_end).astype(jnp.float32),
                   axis=1, keepdims=True)
    texp_ref[...] = jnp.minimum(texp, _E - 1).astype(jnp.int32)


def _ffn_body(texp_ref, padoff_ref, used_ref, rs_ref,
              rte_ref, comb_ref, flat_ref, w1_ref, b1_ref, w2_ref, b2_ref,
              out_ref, gx_s):
    t = pl.program_id(0)

    @pl.when(t == 0)
    def _():
        out_ref[...] = jnp.zeros_like(out_ref)

    @pl.when(t * _TILE < used_ref[0])
    def _():
        e = texp_ref[t]
        r0 = t * _TILE - padoff_ref[e]
        r0f = r0.astype(jnp.float32)
        lane = jax.lax.broadcasted_iota(
            jnp.int32, (1, _TILE), 1).astype(jnp.float32)

        gx_s[...] = jnp.zeros((_TILE, _D), jnp.float32)
        for c in range(_NCHUNK):
            lo = rs_ref[c * _E + e]
            hi = rs_ref[(c + 1) * _E + e]

            @pl.when((lo < r0 + _TILE) & (hi > r0))
            def _(c=c):
                sl = slice(c * _CH, (c + 1) * _CH)
                ei = jax.lax.broadcasted_iota(jnp.int32, (_CH, _E), 1)
                esel = (ei == e).astype(jnp.float32)
                rcol = jnp.sum(rte_ref[sl, :] * esel, axis=1, keepdims=True)
                mt = (rcol == r0f + lane).astype(jnp.float32)
                gx_s[...] += jax.lax.dot_general(
                    mt, flat_ref[sl, :], (((0,), (0,)), ((), ())),
                    preferred_element_type=jnp.float32)

        h = jax.nn.gelu(jnp.dot(gx_s[...], w1_ref[0],
                                preferred_element_type=jnp.float32)
                        + b1_ref[0])
        y = jnp.dot(h, w2_ref[0],
                    preferred_element_type=jnp.float32) + b2_ref[0]

        for c in range(_NCHUNK):
            lo = rs_ref[c * _E + e]
            hi = rs_ref[(c + 1) * _E + e]

            @pl.when((lo < r0 + _TILE) & (hi > r0))
            def _(c=c):
                sl = slice(c * _CH, (c + 1) * _CH)
                ei = jax.lax.broadcasted_iota(jnp.int32, (_CH, _E), 1)
                esel = (ei == e).astype(jnp.float32)
                rcol = jnp.sum(rte_ref[sl, :] * esel, axis=1, keepdims=True)
                wcol = jnp.sum(comb_ref[sl, :] * esel, axis=1, keepdims=True)
                mtw = (rcol == r0f + lane).astype(jnp.float32) * wcol
                out_ref[sl, :] += jnp.dot(mtw, y,
                                          preferred_element_type=jnp.float32)


@jax.jit
def kernel(x, Wg, W1, b1, W2, b2):
    flat = x.reshape(_N, _D)

    router = pl.pallas_call(
        _router_body,
        grid=(1,),
        in_specs=[
            pl.BlockSpec((_N, _D), lambda i: (0, 0)),
            pl.BlockSpec((_D, _E), lambda i: (0, 0)),
        ],
        out_specs=[
            pl.BlockSpec((_N, _E), lambda i: (0, 0)),
            pl.BlockSpec((_N, _E), lambda i: (0, 0)),
            pl.BlockSpec((1, _E), lambda i: (0, 0)),
            pl.BlockSpec((_NTILES, 1), lambda i: (0, 0)),
            pl.BlockSpec((1, 1), lambda i: (0, 0)),
            pl.BlockSpec((_RSROWS, _E), lambda i: (0, 0)),
            pl.BlockSpec((1, 1), lambda i: (0, 0)),
        ],
        out_shape=[
            jax.ShapeDtypeStruct((_N, _E), jnp.float32),
            jax.ShapeDtypeStruct((_N, _E), jnp.float32),
            jax.ShapeDtypeStruct((1, _E), jnp.int32),
            jax.ShapeDtypeStruct((_NTILES, 1), jnp.int32),
            jax.ShapeDtypeStruct((1, 1), jnp.int32),
            jax.ShapeDtypeStruct((_RSROWS, _E), jnp.int32),
            jax.ShapeDtypeStruct((1, 1), jnp.float32),
        ],
    )
    rte, comb, padoff, texp, used, rs, aux = router(flat, Wg)

    grid_spec = pltpu.PrefetchScalarGridSpec(
        num_scalar_prefetch=4,
        grid=(_NTILES,),
        in_specs=[
            pl.BlockSpec((_N, _E), lambda t, te, po, us, rs: (0, 0)),
            pl.BlockSpec((_N, _E), lambda t, te, po, us, rs: (0, 0)),
            pl.BlockSpec((_N, _D), lambda t, te, po, us, rs: (0, 0)),
            pl.BlockSpec((1, _D, _F), lambda t, te, po, us, rs: (te[t], 0, 0)),
            pl.BlockSpec((1, 1, _F), lambda t, te, po, us, rs: (te[t], 0, 0)),
            pl.BlockSpec((1, _F, _D), lambda t, te, po, us, rs: (te[t], 0, 0)),
            pl.BlockSpec((1, 1, _D), lambda t, te, po, us, rs: (te[t], 0, 0)),
        ],
        out_specs=pl.BlockSpec((_N, _D), lambda t, te, po, us, rs: (0, 0)),
        scratch_shapes=[pltpu.VMEM((_TILE, _D), jnp.float32)],
    )
    ffn = pl.pallas_call(
        _ffn_body,
        grid_spec=grid_spec,
        out_shape=jax.ShapeDtypeStruct((_N, _D), jnp.float32),
        compiler_params=pltpu.CompilerParams(
            dimension_semantics=("arbitrary",),
            vmem_limit_bytes=128 * 1024 * 1024),
    )
    out = ffn(texp.reshape(_NTILES), padoff.reshape(_E), used.reshape(1),
              rs.reshape(_RSROWS * _E),
              rte, comb, flat, W1, b1.reshape(_E, 1, _F), W2,
              b2.reshape(_E, 1, _D))

    return out.reshape(_B, _S, _D), aux[0, 0]


# speedup vs baseline: 1.0417x; 1.0282x over previous
import jax
import jax.numpy as jnp
from jax.experimental import pallas as pl
from jax.experimental.pallas import tpu as pltpu

_B, _S, _D = 1, 2048, 768
_E, _K, _F = 8, 2, 2048
_N = _B * _S
_TILE = 256
_NPAD = _N * _K + _E * _TILE
_NTILES = _NPAD // _TILE
_CH = 1024
_NCHUNK = _N // _CH
_RSROWS = 8


def _router_body(flat_ref, wg_ref,
                 rte_ref, comb_ref, padoff_ref, texp_ref, used_ref, rs_ref,
                 aux_ref):
    flat = flat_ref[...]
    logits = jnp.dot(flat, wg_ref[...], preferred_element_type=jnp.float32)

    eio = jax.lax.broadcasted_iota(jnp.int32, (_N, _E), 1)
    m1 = jnp.max(logits, axis=1, keepdims=True)
    i1 = jnp.argmax(logits, axis=1)[:, None]
    oh1 = (eio == i1).astype(jnp.float32)
    masked = jnp.where(oh1 > 0.0, jnp.float32(-1e30), logits)
    m2 = jnp.max(masked, axis=1, keepdims=True)
    i2 = jnp.argmax(masked, axis=1)[:, None]
    oh2 = (eio == i2).astype(jnp.float32)

    w1 = 1.0 / (1.0 + jnp.exp(m2 - m1))
    w2 = 1.0 - w1
    comb_ref[...] = w1 * oh1 + w2 * oh2

    p = jnp.exp(logits - m1)
    p = p / jnp.sum(p, axis=1, keepdims=True)
    sel = oh1 + oh2
    aux_ref[...] = (_E / (_N * _N * _K)) * jnp.sum(
        jnp.sum(p, axis=0, keepdims=True) * jnp.sum(sel, axis=0, keepdims=True),
        axis=1, keepdims=True)

    cnt = jnp.sum(sel, axis=0, keepdims=True)
    ch_r = jax.lax.broadcasted_iota(jnp.int32, (_RSROWS, _N), 0)
    ch_c = jax.lax.broadcasted_iota(jnp.int32, (_RSROWS, _N), 1)
    lb = (ch_c < ch_r * _CH).astype(jnp.float32)
    rs = jax.lax.dot_general(lb, sel, (((1,), (0,)), ((), ())),
                             preferred_element_type=jnp.float32)
    rs_ref[...] = rs.astype(jnp.int32)

    l_r = jax.lax.broadcasted_iota(jnp.int32, (_CH, _CH), 0)
    l_c = jax.lax.broadcasted_iota(jnp.int32, (_CH, _CH), 1)
    l256 = (l_c < l_r).astype(jnp.float32)
    excl_parts = []
    for c in range(_NCHUNK):
        sl = slice(c * _CH, (c + 1) * _CH)
        local = jax.lax.dot_general(l256, sel[sl, :], (((1,), (0,)), ((), ())),
                                    preferred_element_type=jnp.float32)
        excl_parts.append(local + rs[c:c + 1, :])
    excl = jnp.concatenate(excl_parts, axis=0)

    rte_ref[...] = jnp.where(sel > 0.0, excl, jnp.float32(-1.0))

    pc = jnp.ceil(cnt / _TILE) * _TILE
    e_r = jax.lax.broadcasted_iota(jnp.int32, (_E, _E), 0)
    e_c = jax.lax.broadcasted_iota(jnp.int32, (_E, _E), 1)
    ltri_e = (e_r < e_c).astype(jnp.float32)
    padoff = jax.lax.dot_general(pc, ltri_e, (((1,), (0,)), ((), ())),
                                 preferred_element_type=jnp.float32)
    pad_end = padoff + pc
    padoff_ref[...] = padoff.astype(jnp.int32)
    used_ref[...] = jnp.sum(pc, axis=1, keepdims=True).astype(jnp.int32)

    t_starts = jax.lax.broadcasted_iota(
        jnp.int32, (_NTILES, _E), 0).astype(jnp.float32) * _TILE
    texp = jnp.sum((t_starts >= pad_end).astype(jnp.float32),
                   axis=1, keepdims=True)
    texp_ref[...] = jnp.minimum(texp, _E - 1).astype(jnp.int32)


def _ffn_body(texp_ref, padoff_ref, used_ref, rs_ref,
              rte_ref, comb_ref, flat_ref, w1_ref, b1_ref, w2_ref, b2_ref,
              out_ref, gx_s, xb_s):
    t = pl.program_id(0)

    @pl.when(t == 0)
    def _():
        out_ref[...] = jnp.zeros_like(out_ref)
        xb_s[...] = flat_ref[...].astype(jnp.bfloat16)

    @pl.when(t * _TILE < used_ref[0])
    def _():
        e = texp_ref[t]
        r0 = t * _TILE - padoff_ref[e]
        r0f = r0.astype(jnp.float32)
        lane = jax.lax.broadcasted_iota(
            jnp.int32, (1, _TILE), 1).astype(jnp.float32)

        gx_s[...] = jnp.zeros((_TILE, _D), jnp.float32)
        for c in range(_NCHUNK):
            lo = rs_ref[c * _E + e]
            hi = rs_ref[(c + 1) * _E + e]

            @pl.when((lo < r0 + _TILE) & (hi > r0))
            def _(c=c):
                sl = slice(c * _CH, (c + 1) * _CH)
                ei = jax.lax.broadcasted_iota(jnp.int32, (_CH, _E), 1)
                esel = (ei == e).astype(jnp.float32)
                rcol = jnp.sum(rte_ref[sl, :] * esel, axis=1, keepdims=True)
                mt = (rcol == r0f + lane).astype(jnp.bfloat16)
                gx_s[...] += jax.lax.dot_general(
                    mt, xb_s[sl, :], (((0,), (0,)), ((), ())),
                    preferred_element_type=jnp.float32)

        h = jax.nn.gelu(jnp.dot(gx_s[...], w1_ref[0],
                                preferred_element_type=jnp.float32)
                        + b1_ref[0])
        y = jnp.dot(h, w2_ref[0],
                    preferred_element_type=jnp.float32) + b2_ref[0]

        for c in range(_NCHUNK):
            lo = rs_ref[c * _E + e]
            hi = rs_ref[(c + 1) * _E + e]

            @pl.when((lo < r0 + _TILE) & (hi > r0))
            def _(c=c):
                sl = slice(c * _CH, (c + 1) * _CH)
                ei = jax.lax.broadcasted_iota(jnp.int32, (_CH, _E), 1)
                esel = (ei == e).astype(jnp.float32)
                rcol = jnp.sum(rte_ref[sl, :] * esel, axis=1, keepdims=True)
                wcol = jnp.sum(comb_ref[sl, :] * esel, axis=1, keepdims=True)
                mtw = (rcol == r0f + lane).astype(jnp.float32) * wcol
                out_ref[sl, :] += jnp.dot(mtw, y,
                                          preferred_element_type=jnp.float32)


@jax.jit
def kernel(x, Wg, W1, b1, W2, b2):
    flat = x.reshape(_N, _D)

    router = pl.pallas_call(
        _router_body,
        grid=(1,),
        in_specs=[
            pl.BlockSpec((_N, _D), lambda i: (0, 0)),
            pl.BlockSpec((_D, _E), lambda i: (0, 0)),
        ],
        out_specs=[
            pl.BlockSpec((_N, _E), lambda i: (0, 0)),
            pl.BlockSpec((_N, _E), lambda i: (0, 0)),
            pl.BlockSpec((1, _E), lambda i: (0, 0)),
            pl.BlockSpec((_NTILES, 1), lambda i: (0, 0)),
            pl.BlockSpec((1, 1), lambda i: (0, 0)),
            pl.BlockSpec((_RSROWS, _E), lambda i: (0, 0)),
            pl.BlockSpec((1, 1), lambda i: (0, 0)),
        ],
        out_shape=[
            jax.ShapeDtypeStruct((_N, _E), jnp.float32),
            jax.ShapeDtypeStruct((_N, _E), jnp.float32),
            jax.ShapeDtypeStruct((1, _E), jnp.int32),
            jax.ShapeDtypeStruct((_NTILES, 1), jnp.int32),
            jax.ShapeDtypeStruct((1, 1), jnp.int32),
            jax.ShapeDtypeStruct((_RSROWS, _E), jnp.int32),
            jax.ShapeDtypeStruct((1, 1), jnp.float32),
        ],
    )
    rte, comb, padoff, texp, used, rs, aux = router(flat, Wg)

    grid_spec = pltpu.PrefetchScalarGridSpec(
        num_scalar_prefetch=4,
        grid=(_NTILES,),
        in_specs=[
            pl.BlockSpec((_N, _E), lambda t, te, po, us, rs: (0, 0)),
            pl.BlockSpec((_N, _E), lambda t, te, po, us, rs: (0, 0)),
            pl.BlockSpec((_N, _D), lambda t, te, po, us, rs: (0, 0)),
            pl.BlockSpec((1, _D, _F), lambda t, te, po, us, rs: (te[t], 0, 0)),
            pl.BlockSpec((1, 1, _F), lambda t, te, po, us, rs: (te[t], 0, 0)),
            pl.BlockSpec((1, _F, _D), lambda t, te, po, us, rs: (te[t], 0, 0)),
            pl.BlockSpec((1, 1, _D), lambda t, te, po, us, rs: (te[t], 0, 0)),
        ],
        out_specs=pl.BlockSpec((_N, _D), lambda t, te, po, us, rs: (0, 0)),
        scratch_shapes=[pltpu.VMEM((_TILE, _D), jnp.float32),
                        pltpu.VMEM((_N, _D), jnp.bfloat16)],
    )
    ffn = pl.pallas_call(
        _ffn_body,
        grid_spec=grid_spec,
        out_shape=jax.ShapeDtypeStruct((_N, _D), jnp.float32),
        compiler_params=pltpu.CompilerParams(
            dimension_semantics=("arbitrary",),
            vmem_limit_bytes=128 * 1024 * 1024),
    )
    out = ffn(texp.reshape(_NTILES), padoff.reshape(_E), used.reshape(1),
              rs.reshape(_RSROWS * _E),
              rte, comb, flat, W1, b1.reshape(_E, 1, _F), W2,
              b2.reshape(_E, 1, _D))

    return out.reshape(_B, _S, _D), aux[0, 0]
